# baseline (speedup 1.0000x reference)
import jax
import jax.numpy as jnp
from jax import lax
from jax.experimental import pallas as pl

CLS = 1000
N, K = 4096, 26
RB = 512


def _onehot_t(x_ref, o_ref):
    xb = x_ref[...]
    cio = lax.broadcasted_iota(jnp.int32, (1, CLS, RB), 1)
    o_ref[...] = (xb == cio).astype(jnp.int32)


def kernel(x):
    xt = x.T.reshape(K, 1, N)
    out_t = pl.pallas_call(
        _onehot_t,
        grid=(K, N // RB),
        in_specs=[pl.BlockSpec((1, 1, RB), lambda b, j: (b, 0, j))],
        out_specs=pl.BlockSpec((1, CLS, RB), lambda b, j: (b, 0, j)),
        out_shape=jax.ShapeDtypeStruct((K, CLS, N), jnp.int32),
    )(xt)
    return out_t.transpose(2, 0, 1)


# speedup vs baseline: 1.3730x; 1.3730x over previous
import jax
import jax.numpy as jnp
from jax import lax
from jax.experimental import pallas as pl

CLS = 1000
N, K = 4096, 26
RB = 1024


def _onehot_t(x_ref, o_ref):
    xb = x_ref[...]
    cio = lax.broadcasted_iota(jnp.int32, (1, CLS, RB), 1)
    o_ref[...] = (xb == cio).astype(jnp.int32)


def kernel(x):
    xt = x.T.reshape(K, 1, N)
    out_t = pl.pallas_call(
        _onehot_t,
        grid=(K, N // RB),
        in_specs=[pl.BlockSpec((1, 1, RB), lambda b, j: (b, 0, j))],
        out_specs=pl.BlockSpec((1, CLS, RB), lambda b, j: (b, 0, j)),
        out_shape=jax.ShapeDtypeStruct((K, CLS, N), jnp.int32),
    )(xt)
    return out_t.transpose(2, 0, 1)
